# Initial kernel scaffold; baseline (speedup 1.0000x reference)
import jax
import jax.numpy as jnp
from jax.experimental import pallas as pl


def kernel(x, gate_w, W1, b1, W2, b2, Ws1, bs1, Ws2, bs2, route_scale):
    raise NotImplementedError("write your pallas kernel here")


# speedup vs baseline: 1.5199x; 1.5199x over previous
import functools

import jax
import jax.numpy as jnp
from jax.experimental import pallas as pl
from jax.experimental.pallas import tpu as pltpu

_K = 2


def _gelu_exact(v):
    return 0.5 * v * (1.0 + jax.lax.erf(v * (2.0 ** -0.5)))


def _dense_moe_body(x_ref, w_ref, W1_ref, W2_ref, b1_ref, b2_ref, out_ref):
    e = pl.program_id(1)
    x = x_ref[...]
    h = jax.lax.dot_general(x, W1_ref[0], (((1,), (1,)), ((), ())),
                            preferred_element_type=jnp.float32)
    h = h + b1_ref[0]
    h = _gelu_exact(h)
    y = jax.lax.dot_general(h, W2_ref[0], (((1,), (1,)), ((), ())),
                            preferred_element_type=jnp.float32)
    y = y + b2_ref[0]
    contrib = w_ref[0] * y

    @pl.when(e == 0)
    def _():
        out_ref[...] = contrib

    @pl.when(e > 0)
    def _():
        out_ref[...] = out_ref[...] + contrib


def _dense_moe(x2, w3, W1a, W2a, b1a, b2a, *, interpret=False):
    S, D = x2.shape
    E9, F, _ = W1a.shape
    Bt = min(512, S)
    T = S // Bt
    grid = (T, E9)
    return pl.pallas_call(
        _dense_moe_body,
        grid=grid,
        in_specs=[
            pl.BlockSpec((Bt, D), lambda t, e: (t, 0)),
            pl.BlockSpec((1, Bt, 1), lambda t, e: (e, t, 0)),
            pl.BlockSpec((1, F, D), lambda t, e: (e, 0, 0)),
            pl.BlockSpec((1, D, F), lambda t, e: (e, 0, 0)),
            pl.BlockSpec((1, 1, F), lambda t, e: (e, 0, 0)),
            pl.BlockSpec((1, 1, D), lambda t, e: (e, 0, 0)),
        ],
        out_specs=pl.BlockSpec((Bt, D), lambda t, e: (t, 0)),
        out_shape=jax.ShapeDtypeStruct((S, D), jnp.float32),
        compiler_params=pltpu.CompilerParams(
            dimension_semantics=("parallel", "arbitrary"),
        ),
        interpret=interpret,
    )(x2, w3, W1a, W2a, b1a, b2a)


def kernel(x, gate_w, W1, b1, W2, b2, Ws1, bs1, Ws2, bs2, route_scale,
           *, interpret=False):
    original_shape = x.shape
    if x.ndim == 2:
        x = x[:, None, :]
    Bx, Sx, D = x.shape
    E, F, _ = W1.shape

    gate_scores = x @ gate_w.T
    scores = jax.nn.sigmoid(gate_scores) * route_scale
    topk_scores, topk_idx = jax.lax.top_k(scores, _K)
    topk_w = topk_scores / jnp.sum(topk_scores, axis=-1, keepdims=True)

    onehot = jax.nn.one_hot(topk_idx, E, dtype=jnp.float32)
    w_full = jnp.einsum("bske,bsk->bse", onehot, topk_w)
    present = jnp.any(onehot > 0, axis=(0, 1))
    counts = jnp.sum(present.astype(jnp.float32), axis=0)
    expert_usage = counts / jnp.sum(counts)

    S = Bx * Sx
    x2 = x.reshape(S, D)
    w9 = jnp.concatenate(
        [w_full.reshape(S, E), jnp.ones((S, 1), jnp.float32)], axis=1)
    w3 = w9.T[:, :, None]
    W1a = jnp.concatenate([W1, Ws1[None]], axis=0)
    W2a = jnp.concatenate([W2, Ws2[None]], axis=0)
    b1a = jnp.concatenate([b1, bs1[None]], axis=0)[:, None, :]
    b2a = jnp.concatenate([b2, bs2[None]], axis=0)[:, None, :]

    out = _dense_moe(x2, w3, W1a, W2a, b1a, b2a, interpret=interpret)
    output = out.reshape(original_shape)
    return output, expert_usage, topk_idx
